# baseline (speedup 1.0000x reference)
import functools

import jax
import jax.numpy as jnp
from jax import lax
from jax.experimental import pallas as pl
from jax.experimental.pallas import tpu as pltpu
from jax.experimental.pallas import tpu_sc as plsc

_SC_INFO = plsc.get_sparse_core_info()
_NC = _SC_INFO.num_cores
_NS = _SC_INFO.num_subcores
_NW = _NC * _NS

_N_IDX = 64 * 1024
_PER_W = _N_IDX // _NW
_NBUF = 4
_SUB = _PER_W // _NBUF


def _sc_gather(x, idx_flat):
    mesh = plsc.VectorSubcoreMesh(core_axis_name="c", subcore_axis_name="s")

    @functools.partial(
        pl.kernel,
        mesh=mesh,
        out_type=jax.ShapeDtypeStruct((_N_IDX,), jnp.float32),
        scratch_types=(
            [pltpu.VMEM((_SUB,), jnp.int32) for _ in range(_NBUF)]
            + [pltpu.VMEM((_SUB,), jnp.float32) for _ in range(_NBUF)]
            + [
                pltpu.SemaphoreType.DMA,
                pltpu.SemaphoreType.DMA,
                pltpu.SemaphoreType.DMA,
            ]
        ),
    )
    def gather_kernel(x_hbm, idx_hbm, out_hbm, *rest):
        idx_v = rest[:_NBUF]
        rows_v = rest[_NBUF:2 * _NBUF]
        isem, gsem, wsem = rest[2 * _NBUF:]
        wid = lax.axis_index("s") * _NC + lax.axis_index("c")
        base = wid * _PER_W
        idx_loads = [
            pltpu.async_copy(
                idx_hbm.at[pl.ds(base + b * _SUB, _SUB)], idx_v[b], isem)
            for b in range(_NBUF)
        ]
        gathers = [None] * _NBUF
        writes = [None] * _NBUF
        for b in range(_NBUF):
            idx_loads[b].wait()
            gathers[b] = pltpu.async_copy(
                x_hbm.at[idx_v[b]], rows_v[b], gsem)
        for b in range(_NBUF):
            gathers[b].wait()
            writes[b] = pltpu.async_copy(
                rows_v[b], out_hbm.at[pl.ds(base + b * _SUB, _SUB)], wsem)
        for b in range(_NBUF):
            writes[b].wait()

    return gather_kernel(x, idx_flat)


_REP = 64
_BLK_REP = 32


def _tc_broadcast_body(y_ref, o_ref):
    for t in range(_BLK_REP):
        o_ref[t * 64:(t + 1) * 64, :] = y_ref[...]


def _tc_broadcast(y2):
    out2 = pl.pallas_call(
        _tc_broadcast_body,
        grid=(_REP // _BLK_REP,),
        in_specs=[pl.BlockSpec((64, 1024), lambda i: (0, 0))],
        out_specs=pl.BlockSpec((_BLK_REP * 64, 1024), lambda i: (i, 0)),
        out_shape=jax.ShapeDtypeStruct((_REP * 64, 1024), jnp.float32),
    )(y2)
    return out2


def kernel(x, idx_tensor):
    idx_flat = idx_tensor[0].reshape(-1).astype(jnp.int32)
    y = _sc_gather(x, idx_flat)
    out2 = _tc_broadcast(y.reshape(64, 1024))
    return out2.reshape(64, 64, 1024)


# speedup vs baseline: 1.0148x; 1.0148x over previous
import functools

import jax
import jax.numpy as jnp
from jax import lax
from jax.experimental import pallas as pl
from jax.experimental.pallas import tpu as pltpu
from jax.experimental.pallas import tpu_sc as plsc

_SC_INFO = plsc.get_sparse_core_info()
_NC = _SC_INFO.num_cores
_NS = _SC_INFO.num_subcores
_NW = _NC * _NS

_N_IDX = 64 * 1024
_PER_W = _N_IDX // _NW
_NBUF = 4
_SUB = _PER_W // _NBUF


def _sc_gather(x, idx_flat):
    mesh = plsc.VectorSubcoreMesh(core_axis_name="c", subcore_axis_name="s")

    @functools.partial(
        pl.kernel,
        mesh=mesh,
        out_type=jax.ShapeDtypeStruct((_N_IDX,), jnp.float32),
        scratch_types=(
            [pltpu.VMEM((_SUB,), jnp.int32) for _ in range(_NBUF)]
            + [pltpu.VMEM((_SUB,), jnp.float32) for _ in range(_NBUF)]
            + [
                pltpu.SemaphoreType.DMA,
                pltpu.SemaphoreType.DMA,
                pltpu.SemaphoreType.DMA,
            ]
        ),
    )
    def gather_kernel(x_hbm, idx_hbm, out_hbm, *rest):
        idx_v = rest[:_NBUF]
        rows_v = rest[_NBUF:2 * _NBUF]
        isem, gsem, wsem = rest[2 * _NBUF:]
        wid = lax.axis_index("s") * _NC + lax.axis_index("c")
        base = wid * _PER_W
        idx_loads = [
            pltpu.async_copy(
                idx_hbm.at[pl.ds(base + b * _SUB, _SUB)], idx_v[b], isem)
            for b in range(_NBUF)
        ]
        gathers = [None] * _NBUF
        writes = [None] * _NBUF
        for b in range(_NBUF):
            idx_loads[b].wait()
            gathers[b] = pltpu.async_copy(
                x_hbm.at[idx_v[b]], rows_v[b], gsem)
        for b in range(_NBUF):
            gathers[b].wait()
            writes[b] = pltpu.async_copy(
                rows_v[b], out_hbm.at[pl.ds(base + b * _SUB, _SUB)], wsem)
        for b in range(_NBUF):
            writes[b].wait()

    return gather_kernel(x, idx_flat)


_REP = 64
_BLK_REP = 16


def _tc_broadcast_body(y_ref, o_ref):
    for t in range(_BLK_REP):
        o_ref[t * 64:(t + 1) * 64, :] = y_ref[...]


def _tc_broadcast(y2):
    out2 = pl.pallas_call(
        _tc_broadcast_body,
        grid=(_REP // _BLK_REP,),
        in_specs=[pl.BlockSpec((64, 1024), lambda i: (0, 0))],
        out_specs=pl.BlockSpec((_BLK_REP * 64, 1024), lambda i: (i, 0)),
        out_shape=jax.ShapeDtypeStruct((_REP * 64, 1024), jnp.float32),
    )(y2)
    return out2


def kernel(x, idx_tensor):
    idx_flat = idx_tensor[0].reshape(-1).astype(jnp.int32)
    y = _sc_gather(x, idx_flat)
    out2 = _tc_broadcast(y.reshape(64, 1024))
    return out2.reshape(64, 64, 1024)
